# baseline (speedup 1.0000x reference)
import dataclasses
import functools

import ml_dtypes
import numpy as np
import jax
import jax.numpy as jnp
from jax import lax
from jax.experimental import pallas as pl
from jax.experimental.pallas import tpu as pltpu
from jax.experimental.pallas import tpu_sc as plsc

_D = 768
_MAX_LEN = 4096
_NC = 2
_NS = 16
_NW = _NC * _NS
_C = 32
_LANES = 16
_DU = _D // 2


def _positional_encoding_np(max_len: int, d_model: int) -> np.ndarray:
    pos = np.arange(max_len, dtype=np.float32)[:, None]
    div = np.exp(
        np.arange(0, d_model, 2, dtype=np.float32) * (-np.log(10000.0) / d_model)
    )
    pe = np.zeros((max_len, d_model), dtype=np.float32)
    pe[:, 0::2] = np.sin(pos * div)
    pe[:, 1::2] = np.cos(pos * div)
    return pe


def _packed_pe_np(max_len: int, d_model: int) -> np.ndarray:
    pe = _positional_encoding_np(max_len, d_model)
    bits = pe.astype(ml_dtypes.bfloat16)
    bits = bits.view(np.uint16).astype(np.uint32)
    grp = bits.reshape(max_len, d_model // 32, 2, 16)
    return (grp[:, :, 0, :] | (grp[:, :, 1, :] << 16)).reshape(max_len, d_model // 2)


_PE_PACKED = _packed_pe_np(_MAX_LEN, _D)


def kernel(x, token_table):
    batch, seq = x.shape
    n = batch * seq
    idx = x.astype(jnp.int32).reshape(-1)
    pe_packed = jnp.asarray(_PE_PACKED[:seq].reshape(-1))

    rows_per_w = n // _NW
    win = seq // _NW
    nchunks = rows_per_w // _C
    cpspan = win // _C

    mesh = plsc.VectorSubcoreMesh(core_axis_name="c", subcore_axis_name="s")

    cparams = pltpu.CompilerParams()
    if "needs_layout_passes" in pltpu.CompilerParams.__dataclass_fields__:
        cparams = dataclasses.replace(cparams, needs_layout_passes=False)

    @functools.partial(
        pl.kernel,
        mesh=mesh,
        compiler_params=cparams,
        out_type=jax.ShapeDtypeStruct((n, _D), jnp.float32),
        scratch_types=[
            pltpu.VMEM((win * _DU,), jnp.uint32),
            pltpu.VMEM((rows_per_w,), jnp.int32),
            pltpu.VMEM((_C, _D), jnp.float32),
            pltpu.VMEM((_C, _D), jnp.float32),
            pltpu.VMEM((_C, _D), jnp.float32),
            pltpu.SemaphoreType.DMA,
            pltpu.SemaphoreType.DMA,
            pltpu.SemaphoreType.DMA,
            pltpu.SemaphoreType.DMA,
            pltpu.SemaphoreType.DMA,
            pltpu.SemaphoreType.DMA,
            pltpu.SemaphoreType.DMA,
            pltpu.SemaphoreType.DMA,
        ],
    )
    def sc_embed(table_hbm, idx_hbm, pe_hbm, out_hbm,
                 pe_res, idx_all, rows0, rows1, rows2,
                 pesem, isem, g0, g1, g2, o0, o1, o2):
        wid = lax.axis_index("s") * _NC + lax.axis_index("c")
        wbase = wid * win

        pe_cp = pltpu.async_copy(
            pe_hbm.at[pl.ds(wbase * _DU, win * _DU)], pe_res, pesem
        )
        icp = [
            pltpu.async_copy(
                idx_hbm.at[pl.ds(b * seq + wbase, win)],
                idx_all.at[pl.ds(b * win, win)],
                isem,
            )
            for b in range(batch)
        ]

        rows = [rows0, rows1, rows2]
        gsem = [g0, g1, g2]
        osem = [o0, o1, o2]
        gcp = [None, None, None]
        ocp = [None, None, None]

        def flat_off(c):
            return (c // cpspan) * seq + wbase + (c % cpspan) * _C

        def issue_gather(c):
            rb = c % 3
            ioff = (c // cpspan) * win + (c % cpspan) * _C
            gcp[rb] = pltpu.async_copy(
                table_hbm.at[idx_all.at[pl.ds(ioff, _C)]], rows[rb], gsem[rb]
            )

        for cp in icp:
            cp.wait()
        issue_gather(0)
        issue_gather(1)
        pe_cp.wait()

        for c in range(nchunks):
            rb = c % 3
            poff = (c % cpspan) * _C
            gcp[rb].wait()
            row_v = rows[rb]

            @pl.loop(0, _C)
            def _row(r):
                pword = (poff + r) * _DU

                @plsc.parallel_loop(0, _DU, step=_LANES, unroll=8)
                def _col(g):
                    v = pe_res[pl.ds(pword + g, _LANES)]
                    lo = plsc.bitcast(v << jnp.uint32(16), jnp.float32)
                    hi = plsc.bitcast(v & jnp.uint32(0xFFFF0000), jnp.float32)
                    col = g * 2
                    row_v[r, pl.ds(col, _LANES)] = (
                        row_v[r, pl.ds(col, _LANES)] + lo
                    )
                    row_v[r, pl.ds(col + _LANES, _LANES)] = (
                        row_v[r, pl.ds(col + _LANES, _LANES)] + hi
                    )

            ocp[rb] = pltpu.async_copy(
                row_v, out_hbm.at[pl.ds(flat_off(c), _C)], osem[rb]
            )
            if c + 2 < nchunks:
                nb = (c + 2) % 3
                if ocp[nb] is not None:
                    ocp[nb].wait()
                issue_gather(c + 2)

        for rb in range(3):
            if ocp[rb] is not None:
                ocp[rb].wait()

    out = sc_embed(token_table, idx, pe_packed)
    return out.reshape(batch, seq, _D)


# speedup vs baseline: 1.0226x; 1.0226x over previous
import dataclasses
import functools

import ml_dtypes
import numpy as np
import jax
import jax.numpy as jnp
from jax import lax
from jax.experimental import pallas as pl
from jax.experimental.pallas import tpu as pltpu
from jax.experimental.pallas import tpu_sc as plsc

_D = 768
_MAX_LEN = 4096
_NC = 2
_NS = 16
_NW = _NC * _NS
_C = 16
_RING = 6
_LANES = 16
_DU = _D // 2


def _positional_encoding_np(max_len: int, d_model: int) -> np.ndarray:
    pos = np.arange(max_len, dtype=np.float32)[:, None]
    div = np.exp(
        np.arange(0, d_model, 2, dtype=np.float32) * (-np.log(10000.0) / d_model)
    )
    pe = np.zeros((max_len, d_model), dtype=np.float32)
    pe[:, 0::2] = np.sin(pos * div)
    pe[:, 1::2] = np.cos(pos * div)
    return pe


def _packed_pe_np(max_len: int, d_model: int) -> np.ndarray:
    pe = _positional_encoding_np(max_len, d_model)
    bits = pe.astype(ml_dtypes.bfloat16)
    bits = bits.view(np.uint16).astype(np.uint32)
    grp = bits.reshape(max_len, d_model // 32, 2, 16)
    return (grp[:, :, 0, :] | (grp[:, :, 1, :] << 16)).reshape(max_len, d_model // 2)


_PE_PACKED = _packed_pe_np(_MAX_LEN, _D)


def kernel(x, token_table):
    batch, seq = x.shape
    n = batch * seq
    idx = x.astype(jnp.int32).reshape(-1)
    pe_packed = jnp.asarray(_PE_PACKED[:seq].reshape(-1))

    rows_per_w = n // _NW
    win = seq // _NW
    nchunks = rows_per_w // _C
    cpspan = win // _C

    mesh = plsc.VectorSubcoreMesh(core_axis_name="c", subcore_axis_name="s")

    cparams = pltpu.CompilerParams()
    if "needs_layout_passes" in pltpu.CompilerParams.__dataclass_fields__:
        cparams = dataclasses.replace(cparams, needs_layout_passes=False)

    @functools.partial(
        pl.kernel,
        mesh=mesh,
        compiler_params=cparams,
        out_type=jax.ShapeDtypeStruct((n, _D), jnp.float32),
        scratch_types=(
            [
                pltpu.VMEM((win * _DU,), jnp.uint32),
                pltpu.VMEM((rows_per_w,), jnp.int32),
            ]
            + [pltpu.VMEM((_C, _D), jnp.float32)] * _RING
            + [pltpu.SemaphoreType.DMA] * (2 + 2 * _RING)
        ),
    )
    def sc_embed(table_hbm, idx_hbm, pe_hbm, out_hbm,
                 pe_res, idx_all, *rest):
        rows = list(rest[:_RING])
        pesem, isem = rest[_RING], rest[_RING + 1]
        gsem = list(rest[_RING + 2:_RING + 2 + _RING])
        osem = list(rest[_RING + 2 + _RING:])
        wid = lax.axis_index("s") * _NC + lax.axis_index("c")
        wbase = wid * win

        pe_cp = pltpu.async_copy(
            pe_hbm.at[pl.ds(wbase * _DU, win * _DU)], pe_res, pesem
        )
        icp = [
            pltpu.async_copy(
                idx_hbm.at[pl.ds(b * seq + wbase, win)],
                idx_all.at[pl.ds(b * win, win)],
                isem,
            )
            for b in range(batch)
        ]

        gcp = [None] * _RING
        ocp = [None] * _RING

        def flat_off(c):
            return (c // cpspan) * seq + wbase + (c % cpspan) * _C

        def issue_gather(c):
            rb = c % _RING
            ioff = (c // cpspan) * win + (c % cpspan) * _C
            gcp[rb] = pltpu.async_copy(
                table_hbm.at[idx_all.at[pl.ds(ioff, _C)]], rows[rb], gsem[rb]
            )

        for cp in icp:
            cp.wait()
        for c0 in range(_RING - 1):
            issue_gather(c0)
        pe_cp.wait()

        for c in range(nchunks):
            rb = c % _RING
            poff = (c % cpspan) * _C
            gcp[rb].wait()
            row_v = rows[rb]

            @pl.loop(0, _C)
            def _row(r):
                pword = (poff + r) * _DU

                @plsc.parallel_loop(0, _DU, step=_LANES, unroll=8)
                def _col(g):
                    v = pe_res[pl.ds(pword + g, _LANES)]
                    lo = plsc.bitcast(v << jnp.uint32(16), jnp.float32)
                    hi = plsc.bitcast(v & jnp.uint32(0xFFFF0000), jnp.float32)
                    col = g * 2
                    row_v[r, pl.ds(col, _LANES)] = (
                        row_v[r, pl.ds(col, _LANES)] + lo
                    )
                    row_v[r, pl.ds(col + _LANES, _LANES)] = (
                        row_v[r, pl.ds(col + _LANES, _LANES)] + hi
                    )

            ocp[rb] = pltpu.async_copy(
                row_v, out_hbm.at[pl.ds(flat_off(c), _C)], osem[rb]
            )
            nxt = c + _RING - 1
            if nxt < nchunks:
                nb = nxt % _RING
                if ocp[nb] is not None:
                    ocp[nb].wait()
                issue_gather(nxt)

        for rb in range(_RING):
            if ocp[rb] is not None:
                ocp[rb].wait()

    out = sc_embed(token_table, idx, pe_packed)
    return out.reshape(batch, seq, _D)
